# baseline (speedup 1.0000x reference)
import jax
import jax.numpy as jnp
import numpy as np
from jax import lax
from jax.experimental import pallas as pl
from jax.experimental.pallas import tpu as pltpu
from jax.experimental.pallas import tpu_sc as plsc

_N = 10000
_E = 320000
_D = 128
_NCLS = 16

_NC = 2
_NS = 16
_NW = _NC * _NS

_B = 128
_NBW = 80
_CB = 40
_NCHUNK = _NBW // _CB
_EPAD = _NW * _NBW * _B
_NPAD = 10112
_RPW = _NPAD // _NS

_RB = 1000
_DB = 4096

_PERM = np.concatenate([
    np.arange(64).reshape(4, 16) // 16 * 32 + np.arange(16) * 2 + off
    for off in (0, 1)]).reshape(_D)


def _sc_aggregate(feat, srcp, dstp, zeros_d):
    mesh = plsc.VectorSubcoreMesh(core_axis_name="c", subcore_axis_name="s")
    scratch = [
        pltpu.VMEM_SHARED((_NPAD, _D), jnp.float32),
        pltpu.VMEM((_CB, _B), jnp.int32),
        pltpu.VMEM((_CB, _B), jnp.int32),
        pltpu.VMEM((2, _B, _D // 2), jnp.int32),
        pltpu.VMEM((_B, _D), jnp.float32),
        pltpu.SemaphoreType.DMA,
        pltpu.SemaphoreType.DMA,
    ]

    def body(feat_h, srcp_h, dstp_h, zd_h, agg_o,
             acc_s, src_v, dst_v, rows_v, conv_v, gsem0, gsem1):
        c = lax.axis_index("c")
        s = lax.axis_index("s")
        base = (c * _NS + s) * _NBW
        r0 = s * _RPW

        pltpu.sync_copy(zd_h, acc_s.at[pl.ds(r0, _RPW)])
        plsc.subcore_barrier()

        gsems = (gsem0, gsem1)
        hi_mask = jnp.int32(-65536)

        def g_start(j, b):
            pltpu.async_copy(feat_h.at[src_v.at[j]], rows_v.at[b], gsems[b])

        def finish(j, b):
            pltpu.make_async_copy(
                feat_h.at[src_v.at[j]], rows_v.at[b], gsems[b]).wait()

            @plsc.parallel_loop(0, _B, unroll=8)
            def _(r):
                for g in range(_D // 32):
                    w = rows_v[b, r, pl.ds(16 * g, 16)]
                    lo = plsc.bitcast(w << 16, jnp.float32)
                    hi = plsc.bitcast(w & hi_mask, jnp.float32)
                    conv_v[r, pl.ds(16 * g, 16)] = lo
                    conv_v[r, pl.ds(_D // 2 + 16 * g, 16)] = hi

            pltpu.sync_copy(conv_v, acc_s.at[dst_v.at[j]], add=True)

        @pl.loop(0, _NCHUNK)
        def _(ci):
            gbase = base + ci * _CB
            pltpu.sync_copy(srcp_h.at[pl.ds(gbase, _CB)], src_v)
            pltpu.sync_copy(dstp_h.at[pl.ds(gbase, _CB)], dst_v)

            g_start(0, 0)
            g_start(1, 1)

            @pl.loop(0, _CB - 2, step=2)
            def _(j):
                finish(j, 0)
                g_start(j + 2, 0)
                finish(j + 1, 1)
                g_start(j + 3, 1)

            finish(_CB - 2, 0)
            finish(_CB - 1, 1)

        plsc.subcore_barrier()
        pltpu.sync_copy(acc_s.at[pl.ds(r0, _RPW)],
                        agg_o.at[c, pl.ds(r0, _RPW)])

    fn = pl.kernel(body,
                   out_type=jax.ShapeDtypeStruct((_NC, _NPAD, _D),
                                                 jnp.float32),
                   mesh=mesh, scratch_types=scratch,
                   compiler_params=pltpu.CompilerParams(
                       use_tc_tiling_on_sc=False,
                       needs_layout_passes=False))
    return fn(feat, srcp, dstp, zeros_d)


def _deg_body(dstr_ref, dstc_ref, o_ref):
    hi = dstr_ref[...] >> 7
    lo = dstc_ref[...] & 127
    oht = (hi == lax.broadcasted_iota(jnp.int32, (_NPAD // 128, _DB), 0)
           ).astype(jnp.bfloat16)
    ol = (lo == lax.broadcasted_iota(jnp.int32, (_DB, 128), 1)
          ).astype(jnp.bfloat16)
    p = jnp.dot(oht, ol, preferred_element_type=jnp.float32)

    @pl.when(pl.program_id(0) == 0)
    def _():
        o_ref[...] = jnp.zeros_like(o_ref)

    o_ref[...] += p


def _tc_degrees(dst_row, dst_col):
    return pl.pallas_call(
        _deg_body,
        grid=(_EPAD // _DB,),
        in_specs=[pl.BlockSpec((1, _DB), lambda i: (0, i)),
                  pl.BlockSpec((_DB, 1), lambda i: (i, 0))],
        out_specs=pl.BlockSpec((_NPAD // 128, 128), lambda i: (0, 0)),
        out_shape=jax.ShapeDtypeStruct((_NPAD // 128, 128), jnp.float32),
    )(dst_row, dst_col)


def _dot(a, b):
    return jnp.dot(a, b, preferred_element_type=jnp.float32)


def _neigh_mean(agg_ref, deg_ref):
    agg = agg_ref[0] + agg_ref[1]
    return agg * (1.0 / jnp.maximum(deg_ref[...], 1.0))


def _tc_layer0_body(x_ref, agg_ref, deg_ref, ws_ref, wn_ref, b_ref, o_ref):
    hn = _neigh_mean(agg_ref, deg_ref)
    o = _dot(x_ref[...], ws_ref[...]) + _dot(hn, wn_ref[...]) + b_ref[...]
    o_ref[...] = jnp.maximum(o, 0.0)


def _tc_layer1_body(h1_ref, agg_ref, deg_ref, ws_ref, wn_ref, b_ref, wpp_ref,
                    o_ref):
    hn = _neigh_mean(agg_ref, deg_ref)
    h1 = h1_ref[...]
    h2 = _dot(h1, ws_ref[...]) + _dot(hn, wn_ref[...]) + b_ref[...]
    o_ref[...] = (_dot(jnp.maximum(h2, 0.0), wpp_ref[0:_D, :])
                  + _dot(jnp.maximum(h1, 0.0), wpp_ref[_D:, :]))


def _tc_layer0(x, agg, deg, W_self, b, W_neigh):
    return pl.pallas_call(
        _tc_layer0_body,
        grid=(_N // _RB,),
        in_specs=[
            pl.BlockSpec((_RB, _D), lambda i: (i, 0)),
            pl.BlockSpec((_NC, _RB, _D), lambda i: (0, i, 0)),
            pl.BlockSpec((_RB, 1), lambda i: (i, 0)),
            pl.BlockSpec((_D, _D), lambda i: (0, 0)),
            pl.BlockSpec((_D, _D), lambda i: (0, 0)),
            pl.BlockSpec((1, _D), lambda i: (0, 0)),
        ],
        out_specs=pl.BlockSpec((_RB, _D), lambda i: (i, 0)),
        out_shape=jax.ShapeDtypeStruct((_N, _D), jnp.float32),
    )(x, agg, deg, W_self, W_neigh, b)


def _tc_layer1(h1, agg, deg, W_self, b, W_neigh, W_pp):
    return pl.pallas_call(
        _tc_layer1_body,
        grid=(_N // _RB,),
        in_specs=[
            pl.BlockSpec((_RB, _D), lambda i: (i, 0)),
            pl.BlockSpec((_NC, _RB, _D), lambda i: (0, i, 0)),
            pl.BlockSpec((_RB, 1), lambda i: (i, 0)),
            pl.BlockSpec((_D, _D), lambda i: (0, 0)),
            pl.BlockSpec((_D, _D), lambda i: (0, 0)),
            pl.BlockSpec((1, _D), lambda i: (0, 0)),
            pl.BlockSpec((2 * _D, _NCLS), lambda i: (0, 0)),
        ],
        out_specs=pl.BlockSpec((_RB, _NCLS), lambda i: (i, 0)),
        out_shape=jax.ShapeDtypeStruct((_N, _NCLS), jnp.float32),
    )(h1, agg, deg, W_self, W_neigh, b, W_pp)


def kernel(x, edge_index, W_neigh0, W_self0, b_self0, W_neigh1, W_self1,
           b_self1, W_pp):
    src = edge_index[0]
    dst = edge_index[1]
    pad = _EPAD - _E
    srcp = jnp.concatenate(
        [src, jnp.zeros((pad,), jnp.int32)]).reshape(_EPAD // _B, _B)
    dstp = jnp.concatenate(
        [dst, jnp.full((pad,), _N, jnp.int32)]).reshape(_EPAD // _B, _B)
    zeros_d = jnp.zeros((_RPW, _D), jnp.float32)

    deg = _tc_degrees(dstp.reshape(1, _EPAD),
                      dstp.reshape(_EPAD, 1)).reshape(_NPAD, 1)
    def _pack(a):
        return jax.lax.bitcast_convert_type(
            a.astype(jnp.bfloat16).reshape(_N, _D // 2, 2), jnp.int32)

    agg0 = _sc_aggregate(_pack(x), srcp, dstp, zeros_d)
    h1 = _tc_layer0(x, agg0, deg, W_self0, b_self0.reshape(1, _D),
                    W_neigh0[_PERM])
    agg1 = _sc_aggregate(_pack(h1), srcp, dstp, zeros_d)
    return _tc_layer1(h1, agg1, deg, W_self1, b_self1.reshape(1, _D),
                      W_neigh1[_PERM], W_pp)


# speedup vs baseline: 1.0010x; 1.0010x over previous
import jax
import jax.numpy as jnp
import numpy as np
from jax import lax
from jax.experimental import pallas as pl
from jax.experimental.pallas import tpu as pltpu
from jax.experimental.pallas import tpu_sc as plsc

_N = 10000
_E = 320000
_D = 128
_NCLS = 16

_NC = 2
_NS = 16
_NW = _NC * _NS

_B = 128
_NBW = 80
_CB = 40
_NCHUNK = _NBW // _CB
_EPAD = _NW * _NBW * _B
_NPAD = 10112
_RPW = _NPAD // _NS

_RB = 1000
_DB = 4096

_PERM = np.concatenate([
    np.arange(64).reshape(4, 16) // 16 * 32 + np.arange(16) * 2 + off
    for off in (0, 1)]).reshape(_D)


def _sc_aggregate(feat, srcp, dstp, zeros_d):
    mesh = plsc.VectorSubcoreMesh(core_axis_name="c", subcore_axis_name="s")
    scratch = [
        pltpu.VMEM_SHARED((_NPAD, _D), jnp.float32),
        pltpu.VMEM((_CB, _B), jnp.int32),
        pltpu.VMEM((_CB, _B), jnp.int32),
        pltpu.VMEM((2, _B, _D // 2), jnp.int32),
        pltpu.VMEM((_B, _D), jnp.float32),
        pltpu.SemaphoreType.DMA,
        pltpu.SemaphoreType.DMA,
    ]

    def body(feat_h, srcp_h, dstp_h, zd_h, agg_o,
             acc_s, src_v, dst_v, rows_v, conv_v, gsem0, gsem1):
        c = lax.axis_index("c")
        s = lax.axis_index("s")
        base = (c * _NS + s) * _NBW
        r0 = s * _RPW

        pltpu.sync_copy(zd_h, acc_s.at[pl.ds(r0, _RPW)])
        plsc.subcore_barrier()

        gsems = (gsem0, gsem1)
        hi_mask = jnp.int32(-65536)

        def g_start(j, b):
            pltpu.async_copy(feat_h.at[src_v.at[j]], rows_v.at[b], gsems[b])

        def finish(j, b):
            pltpu.make_async_copy(
                feat_h.at[src_v.at[j]], rows_v.at[b], gsems[b]).wait()

            @plsc.parallel_loop(0, _B, unroll=4)
            def _(r):
                for g in range(_D // 32):
                    w = rows_v[b, r, pl.ds(16 * g, 16)]
                    lo = plsc.bitcast(w << 16, jnp.float32)
                    hi = plsc.bitcast(w & hi_mask, jnp.float32)
                    conv_v[r, pl.ds(16 * g, 16)] = lo
                    conv_v[r, pl.ds(_D // 2 + 16 * g, 16)] = hi

            pltpu.sync_copy(conv_v, acc_s.at[dst_v.at[j]], add=True)

        @pl.loop(0, _NCHUNK)
        def _(ci):
            gbase = base + ci * _CB
            pltpu.sync_copy(srcp_h.at[pl.ds(gbase, _CB)], src_v)
            pltpu.sync_copy(dstp_h.at[pl.ds(gbase, _CB)], dst_v)

            g_start(0, 0)
            g_start(1, 1)

            @pl.loop(0, _CB - 2, step=2)
            def _(j):
                finish(j, 0)
                g_start(j + 2, 0)
                finish(j + 1, 1)
                g_start(j + 3, 1)

            finish(_CB - 2, 0)
            finish(_CB - 1, 1)

        plsc.subcore_barrier()
        pltpu.sync_copy(acc_s.at[pl.ds(r0, _RPW)],
                        agg_o.at[c, pl.ds(r0, _RPW)])

    fn = pl.kernel(body,
                   out_type=jax.ShapeDtypeStruct((_NC, _NPAD, _D),
                                                 jnp.float32),
                   mesh=mesh, scratch_types=scratch,
                   compiler_params=pltpu.CompilerParams(
                       use_tc_tiling_on_sc=False,
                       needs_layout_passes=False))
    return fn(feat, srcp, dstp, zeros_d)


def _deg_body(dstr_ref, dstc_ref, o_ref):
    hi = dstr_ref[...] >> 7
    lo = dstc_ref[...] & 127
    oht = (hi == lax.broadcasted_iota(jnp.int32, (_NPAD // 128, _DB), 0)
           ).astype(jnp.bfloat16)
    ol = (lo == lax.broadcasted_iota(jnp.int32, (_DB, 128), 1)
          ).astype(jnp.bfloat16)
    p = jnp.dot(oht, ol, preferred_element_type=jnp.float32)

    @pl.when(pl.program_id(0) == 0)
    def _():
        o_ref[...] = jnp.zeros_like(o_ref)

    o_ref[...] += p


def _tc_degrees(dst_row, dst_col):
    return pl.pallas_call(
        _deg_body,
        grid=(_EPAD // _DB,),
        in_specs=[pl.BlockSpec((1, _DB), lambda i: (0, i)),
                  pl.BlockSpec((_DB, 1), lambda i: (i, 0))],
        out_specs=pl.BlockSpec((_NPAD // 128, 128), lambda i: (0, 0)),
        out_shape=jax.ShapeDtypeStruct((_NPAD // 128, 128), jnp.float32),
    )(dst_row, dst_col)


def _dot(a, b):
    return jnp.dot(a, b, preferred_element_type=jnp.float32)


def _neigh_mean(agg_ref, deg_ref):
    agg = agg_ref[0] + agg_ref[1]
    return agg * (1.0 / jnp.maximum(deg_ref[...], 1.0))


def _tc_layer0_body(x_ref, agg_ref, deg_ref, ws_ref, wn_ref, b_ref, o_ref):
    hn = _neigh_mean(agg_ref, deg_ref)
    o = _dot(x_ref[...], ws_ref[...]) + _dot(hn, wn_ref[...]) + b_ref[...]
    o_ref[...] = jnp.maximum(o, 0.0)


def _tc_layer1_body(h1_ref, agg_ref, deg_ref, ws_ref, wn_ref, b_ref, wpp_ref,
                    o_ref):
    hn = _neigh_mean(agg_ref, deg_ref)
    h1 = h1_ref[...]
    h2 = _dot(h1, ws_ref[...]) + _dot(hn, wn_ref[...]) + b_ref[...]
    o_ref[...] = (_dot(jnp.maximum(h2, 0.0), wpp_ref[0:_D, :])
                  + _dot(jnp.maximum(h1, 0.0), wpp_ref[_D:, :]))


def _tc_layer0(x, agg, deg, W_self, b, W_neigh):
    return pl.pallas_call(
        _tc_layer0_body,
        grid=(_N // _RB,),
        in_specs=[
            pl.BlockSpec((_RB, _D), lambda i: (i, 0)),
            pl.BlockSpec((_NC, _RB, _D), lambda i: (0, i, 0)),
            pl.BlockSpec((_RB, 1), lambda i: (i, 0)),
            pl.BlockSpec((_D, _D), lambda i: (0, 0)),
            pl.BlockSpec((_D, _D), lambda i: (0, 0)),
            pl.BlockSpec((1, _D), lambda i: (0, 0)),
        ],
        out_specs=pl.BlockSpec((_RB, _D), lambda i: (i, 0)),
        out_shape=jax.ShapeDtypeStruct((_N, _D), jnp.float32),
    )(x, agg, deg, W_self, W_neigh, b)


def _tc_layer1(h1, agg, deg, W_self, b, W_neigh, W_pp):
    return pl.pallas_call(
        _tc_layer1_body,
        grid=(_N // _RB,),
        in_specs=[
            pl.BlockSpec((_RB, _D), lambda i: (i, 0)),
            pl.BlockSpec((_NC, _RB, _D), lambda i: (0, i, 0)),
            pl.BlockSpec((_RB, 1), lambda i: (i, 0)),
            pl.BlockSpec((_D, _D), lambda i: (0, 0)),
            pl.BlockSpec((_D, _D), lambda i: (0, 0)),
            pl.BlockSpec((1, _D), lambda i: (0, 0)),
            pl.BlockSpec((2 * _D, _NCLS), lambda i: (0, 0)),
        ],
        out_specs=pl.BlockSpec((_RB, _NCLS), lambda i: (i, 0)),
        out_shape=jax.ShapeDtypeStruct((_N, _NCLS), jnp.float32),
    )(h1, agg, deg, W_self, W_neigh, b, W_pp)


def kernel(x, edge_index, W_neigh0, W_self0, b_self0, W_neigh1, W_self1,
           b_self1, W_pp):
    src = edge_index[0]
    dst = edge_index[1]
    pad = _EPAD - _E
    srcp = jnp.concatenate(
        [src, jnp.zeros((pad,), jnp.int32)]).reshape(_EPAD // _B, _B)
    dstp = jnp.concatenate(
        [dst, jnp.full((pad,), _N, jnp.int32)]).reshape(_EPAD // _B, _B)
    zeros_d = jnp.zeros((_RPW, _D), jnp.float32)

    deg = _tc_degrees(dstp.reshape(1, _EPAD),
                      dstp.reshape(_EPAD, 1)).reshape(_NPAD, 1)
    def _pack(a):
        return jax.lax.bitcast_convert_type(
            a.astype(jnp.bfloat16).reshape(_N, _D // 2, 2), jnp.int32)

    agg0 = _sc_aggregate(_pack(x), srcp, dstp, zeros_d)
    h1 = _tc_layer0(x, agg0, deg, W_self0, b_self0.reshape(1, _D),
                    W_neigh0[_PERM])
    agg1 = _sc_aggregate(_pack(h1), srcp, dstp, zeros_d)
    return _tc_layer1(h1, agg1, deg, W_self1, b_self1.reshape(1, _D),
                      W_neigh1[_PERM], W_pp)
